# Initial kernel scaffold; baseline (speedup 1.0000x reference)
import jax
import jax.numpy as jnp
from jax.experimental import pallas as pl


def kernel(x, binary, scale, bias):
    raise NotImplementedError("write your pallas kernel here")


# speedup vs baseline: 725.7606x; 725.7606x over previous
import functools

import jax
import jax.numpy as jnp
from jax.experimental import pallas as pl


def _bq_matmul_kernel(x_ref, binary_ref, scale_ref, bias_ref, out_ref):
    nbits, g, nf = binary_ref.shape
    shifts = 7 - jax.lax.broadcasted_iota(jnp.int32, (1, 8, 1), 1)
    w = None
    for b in range(nbits):
        byte = binary_ref[b]
        bits = (byte[:, None, :] >> shifts) & 1
        signed = (2 * bits - 1).astype(jnp.float32)
        wb = signed * scale_ref[b][None, None, :]
        w = wb if w is None else w + wb
    weff = w.reshape(g * 8, nf)
    acc = jnp.dot(x_ref[...], weff,
                  preferred_element_type=jnp.float32,
                  precision=jax.lax.Precision.HIGHEST)
    out_ref[...] = acc + bias_ref[...]


@functools.partial(jax.jit, static_argnames=())
def kernel(x, binary, scale, bias):
    size_out = x.shape[:-1] + (bias.shape[-1],)
    x2 = x.reshape(-1, x.shape[-1])
    t, nx = x2.shape
    nbits = scale.shape[1]
    nf = scale.shape[2]
    g = nx // 8
    binary3 = binary.reshape(nbits, g, nf)
    scale2 = scale.reshape(nbits, nf)
    bias2 = bias.reshape(1, nf)
    out = pl.pallas_call(
        _bq_matmul_kernel,
        out_shape=jax.ShapeDtypeStruct((t, nf), jnp.float32),
    )(x2, binary3, scale2, bias2)
    return out.reshape(size_out)


# speedup vs baseline: 1230.4062x; 1.6953x over previous
import functools

import jax
import jax.numpy as jnp
from jax.experimental import pallas as pl


def _bq_matmul_kernel(x_ref, binary_ref, scale_ref, bias_ref, out_ref):
    nbits, g, nf = binary_ref.shape
    shifts = 7 - jax.lax.broadcasted_iota(jnp.int32, (1, 8, 1), 1)
    acc = None
    for b in range(nbits):
        byte = binary_ref[b]
        bits = (byte[:, None, :] >> shifts) & 1
        fb = bits.astype(jnp.float32) * scale_ref[b][None, None, :]
        acc = fb if acc is None else acc + fb
    csum = jnp.sum(scale_ref[...], axis=0)
    w = 2.0 * acc - csum[None, None, :]
    weff = w.reshape(g * 8, nf).astype(jnp.bfloat16)
    xb = x_ref[...].astype(jnp.bfloat16)
    out = jnp.dot(xb, weff, preferred_element_type=jnp.float32)
    out_ref[...] = out + bias_ref[...]


@functools.partial(jax.jit, static_argnames=())
def kernel(x, binary, scale, bias):
    size_out = x.shape[:-1] + (bias.shape[-1],)
    x2 = x.reshape(-1, x.shape[-1])
    t, nx = x2.shape
    nbits = scale.shape[1]
    nf = scale.shape[2]
    g = nx // 8
    binary3 = binary.reshape(nbits, g, nf)
    scale2 = scale.reshape(nbits, nf)
    bias2 = bias.reshape(1, nf)
    out = pl.pallas_call(
        _bq_matmul_kernel,
        out_shape=jax.ShapeDtypeStruct((t, nf), jnp.float32),
    )(x2, binary3, scale2, bias2)
    return out.reshape(size_out)
